# baseline (speedup 1.0000x reference)
import functools

import jax
import jax.numpy as jnp
from jax import lax
from jax.experimental import pallas as pl
from jax.experimental.pallas import tpu as pltpu
from jax.experimental.pallas import tpu_sc as plsc

B = 100000
K = 128
L = 16
NC = 2
NS = 16
NW = NC * NS
ROWS_PER_TILE = B // NW
CHUNK_ROWS = 25
NCHUNK = ROWS_PER_TILE // CHUNK_ROWS
CHUNK_ELEMS = CHUNK_ROWS * K
VPR = K // L
UNROLL = 12


def _sc_body(w_hbm, z_hbm, u_hbm, t_hbm, out_hbm,
             wb0, zb0, ub0, tb0, wb1, zb1, ub1, tb1, ob0, ob1,
             sem0, sem1, osem0, osem1):
    c = lax.axis_index("c")
    s = lax.axis_index("s")
    wid = s * NC + c
    tile_base = wid * ROWS_PER_TILE * K
    bufs = ((wb0, zb0, ub0, tb0, sem0), (wb1, zb1, ub1, tb1, sem1))
    obufs = ((ob0, osem0), (ob1, osem1))

    def in_copies(ci, p):
        base = tile_base + ci * CHUNK_ELEMS
        wb, zb, ub, tb, sem = bufs[p]
        return (
            pltpu.make_async_copy(w_hbm.at[pl.ds(base, CHUNK_ELEMS)], wb, sem),
            pltpu.make_async_copy(z_hbm.at[pl.ds(base, CHUNK_ELEMS)], zb, sem),
            pltpu.make_async_copy(u_hbm.at[pl.ds(base, CHUNK_ELEMS)], ub, sem),
            pltpu.make_async_copy(t_hbm.at[pl.ds(base, CHUNK_ELEMS)], tb, sem),
        )

    def start_in(ci, p):
        for h in in_copies(ci, p):
            h.start()

    def wait_in(ci, p):
        for h in in_copies(ci, p):
            h.wait()

    def out_copy(ci, p):
        base = tile_base + ci * CHUNK_ELEMS
        ob, osem = obufs[p]
        return pltpu.make_async_copy(ob, out_hbm.at[pl.ds(base, CHUNK_ELEMS)], osem)

    def one_row(ro, wb, zb, ub, tb, obuf):
        total = jnp.float32(0.0)
        bp = []
        for j in range(VPR):
            v = wb[pl.ds(ro + j * L, L)] + jnp.float32(1e-5)
            cs = plsc.cumsum(v) + total
            wb[pl.ds(ro + j * L, L)] = cs
            total = total + jnp.sum(v)
            bp.append(total)

        ro_vec = jnp.full((L,), ro, jnp.int32)
        i64 = jnp.int32(64)
        i32 = jnp.int32(32)
        i16 = jnp.int32(16)
        i0 = jnp.int32(0)
        bpv = [jnp.full((L,), bp[k]) for k in range(7)]
        uvs = [ub[pl.ds(ro + j * L, L)] * total for j in range(VPR)]
        poss = []
        for j in range(VPR):
            uv = uvs[j]
            m3 = bpv[3] <= uv
            t1 = jnp.where(m3, bpv[5], bpv[1])
            m1 = t1 <= uv
            t2 = jnp.where(
                m1, jnp.where(m3, bpv[6], bpv[2]), jnp.where(m3, bpv[4], bpv[0])
            )
            m0 = t2 <= uv
            p16 = (
                jnp.where(m3, i64, i0)
                + jnp.where(m1, i32, i0)
                + jnp.where(m0, i16, i0)
            )
            poss.append(ro_vec + p16)
        for step in (8, 4, 2, 1):
            istep = jnp.int32(step)
            vals = [plsc.load_gather(wb, [poss[j] + (step - 1)]) for j in range(VPR)]
            poss = [
                poss[j] + jnp.where(vals[j] <= uvs[j], istep, i0)
                for j in range(VPR)
            ]
        ro_hi = ro_vec + jnp.int32(K - 1)
        gas = [
            plsc.load_gather(zb, [jnp.maximum(poss[j] - 1, ro_vec)])
            for j in range(VPR)
        ]
        gbs = [plsc.load_gather(zb, [poss[j]]) for j in range(VPR)]
        gcs = [
            plsc.load_gather(zb, [jnp.minimum(poss[j] + 1, ro_hi)])
            for j in range(VPR)
        ]
        for j in range(VPR):
            tv = tb[pl.ds(ro + j * L, L)]
            obuf[pl.ds(ro + j * L, L)] = jnp.float32(0.5) * (
                gas[j] + gbs[j] + tv * (gcs[j] - gas[j])
            )

    def compute_chunk(ci, p):
        wb, zb, ub, tb, _ = bufs[p]
        obuf = obufs[p][0]

        @plsc.parallel_loop(0, CHUNK_ROWS, step=1, unroll=UNROLL)
        def row_body(r):
            one_row(r * K, wb, zb, ub, tb, obuf)
        out_copy(ci, p).start()

    start_in(0, 0)

    def chunk_pair(g, carry):
        c0 = g * 2
        start_in(c0 + 1, 1)
        wait_in(c0, 0)

        @pl.when(g > 0)
        def _():
            out_copy(c0 - 2, 0).wait()

        compute_chunk(c0, 0)
        start_in(c0 + 2, 0)
        wait_in(c0 + 1, 1)

        @pl.when(g > 0)
        def _():
            out_copy(c0 - 1, 1).wait()

        compute_chunk(c0 + 1, 1)
        return carry

    lax.fori_loop(0, (NCHUNK - 1) // 2, chunk_pair, 0)
    wait_in(NCHUNK - 1, 0)
    out_copy(NCHUNK - 3, 0).wait()
    compute_chunk(NCHUNK - 1, 0)
    out_copy(NCHUNK - 2, 1).wait()
    out_copy(NCHUNK - 1, 0).wait()


@functools.partial(jax.jit, static_argnames=())
def _sc_call(weights, z_samp, u, interval_interp):
    mesh = plsc.VectorSubcoreMesh(
        core_axis_name="c", subcore_axis_name="s", num_cores=NC, num_subcores=NS
    )
    fb = jnp.float32
    fn = pl.kernel(
        _sc_body,
        out_type=jax.ShapeDtypeStruct((B * K,), fb),
        mesh=mesh,
        scratch_types=[pltpu.VMEM((CHUNK_ELEMS,), fb) for _ in range(10)]
        + [pltpu.SemaphoreType.DMA for _ in range(4)],
        compiler_params=pltpu.CompilerParams(needs_layout_passes=False),
    )
    return fn(weights, z_samp, u, interval_interp)


def kernel(rays, weights, z_samp, u, interval_interp):
    del rays
    out = _sc_call(
        weights.reshape(-1),
        z_samp.reshape(-1),
        u.reshape(-1),
        interval_interp.reshape(-1),
    )
    return out.reshape(B, K)


# speedup vs baseline: 1.8471x; 1.8471x over previous
import functools

import jax
import jax.numpy as jnp
from jax import lax
from jax.experimental import pallas as pl
from jax.experimental.pallas import tpu as pltpu
from jax.experimental.pallas import tpu_sc as plsc

B = 100000
K = 128
L = 16
NC = 2
NS = 16
NW = NC * NS
ROWS_PER_TILE = B // NW
CHUNK_ROWS = 25
NCHUNK = ROWS_PER_TILE // CHUNK_ROWS
CHUNK_ELEMS = CHUNK_ROWS * K
VPR = K // L
UNROLL = 5


def _sc_body(w_hbm, z_hbm, u_hbm, t_hbm, out_hbm,
             wb0, zb0, ub0, tb0, wb1, zb1, ub1, tb1, ob0, ob1,
             sem0, sem1, osem0, osem1):
    c = lax.axis_index("c")
    s = lax.axis_index("s")
    wid = s * NC + c
    tile_base = wid * ROWS_PER_TILE * K
    bufs = ((wb0, zb0, ub0, tb0, sem0), (wb1, zb1, ub1, tb1, sem1))
    obufs = ((ob0, osem0), (ob1, osem1))

    def in_copies(ci, p):
        base = tile_base + ci * CHUNK_ELEMS
        wb, zb, ub, tb, sem = bufs[p]
        return (
            pltpu.make_async_copy(w_hbm.at[pl.ds(base, CHUNK_ELEMS)], wb, sem),
            pltpu.make_async_copy(z_hbm.at[pl.ds(base, CHUNK_ELEMS)], zb, sem),
            pltpu.make_async_copy(u_hbm.at[pl.ds(base, CHUNK_ELEMS)], ub, sem),
            pltpu.make_async_copy(t_hbm.at[pl.ds(base, CHUNK_ELEMS)], tb, sem),
        )

    def start_in(ci, p):
        for h in in_copies(ci, p):
            h.start()

    def wait_in(ci, p):
        for h in in_copies(ci, p):
            h.wait()

    def out_copy(ci, p):
        base = tile_base + ci * CHUNK_ELEMS
        ob, osem = obufs[p]
        return pltpu.make_async_copy(ob, out_hbm.at[pl.ds(base, CHUNK_ELEMS)], osem)

    def one_row(ro, wb, zb, ub, tb, obuf):
        total = jnp.float32(0.0)
        bp = []
        for j in range(VPR):
            v = wb[pl.ds(ro + j * L, L)] + jnp.float32(1e-5)
            cs = plsc.cumsum(v) + total
            wb[pl.ds(ro + j * L, L)] = cs
            total = total + jnp.sum(v)
            bp.append(total)

        ro_vec = jnp.full((L,), ro, jnp.int32)
        i64 = jnp.int32(64)
        i32 = jnp.int32(32)
        i16 = jnp.int32(16)
        i0 = jnp.int32(0)
        bpv = [jnp.full((L,), bp[k]) for k in range(7)]
        uvs = [ub[pl.ds(ro + j * L, L)] * total for j in range(VPR)]
        poss = []
        for j in range(VPR):
            uv = uvs[j]
            m3 = bpv[3] <= uv
            t1 = jnp.where(m3, bpv[5], bpv[1])
            m1 = t1 <= uv
            t2 = jnp.where(
                m1, jnp.where(m3, bpv[6], bpv[2]), jnp.where(m3, bpv[4], bpv[0])
            )
            m0 = t2 <= uv
            p16 = (
                jnp.where(m3, i64, i0)
                + jnp.where(m1, i32, i0)
                + jnp.where(m0, i16, i0)
            )
            poss.append(ro_vec + p16)
        for step in (8, 4, 2, 1):
            istep = jnp.int32(step)
            vals = [plsc.load_gather(wb, [poss[j] + (step - 1)]) for j in range(VPR)]
            poss = [
                poss[j] + jnp.where(vals[j] <= uvs[j], istep, i0)
                for j in range(VPR)
            ]
        ro_hi = ro_vec + jnp.int32(K - 1)
        gas = [
            plsc.load_gather(zb, [jnp.maximum(poss[j] - 1, ro_vec)])
            for j in range(VPR)
        ]
        gbs = [plsc.load_gather(zb, [poss[j]]) for j in range(VPR)]
        gcs = [
            plsc.load_gather(zb, [jnp.minimum(poss[j] + 1, ro_hi)])
            for j in range(VPR)
        ]
        for j in range(VPR):
            tv = tb[pl.ds(ro + j * L, L)]
            obuf[pl.ds(ro + j * L, L)] = jnp.float32(0.5) * (
                gas[j] + gbs[j] + tv * (gcs[j] - gas[j])
            )

    def compute_chunk(ci, p):
        wb, zb, ub, tb, _ = bufs[p]
        obuf = obufs[p][0]

        @plsc.parallel_loop(0, CHUNK_ROWS, step=1, unroll=UNROLL)
        def row_body(r):
            one_row(r * K, wb, zb, ub, tb, obuf)
        out_copy(ci, p).start()

    start_in(0, 0)

    def chunk_pair(g, carry):
        c0 = g * 2
        start_in(c0 + 1, 1)
        wait_in(c0, 0)

        @pl.when(g > 0)
        def _():
            out_copy(c0 - 2, 0).wait()

        compute_chunk(c0, 0)
        start_in(c0 + 2, 0)
        wait_in(c0 + 1, 1)

        @pl.when(g > 0)
        def _():
            out_copy(c0 - 1, 1).wait()

        compute_chunk(c0 + 1, 1)
        return carry

    lax.fori_loop(0, (NCHUNK - 1) // 2, chunk_pair, 0)
    wait_in(NCHUNK - 1, 0)
    out_copy(NCHUNK - 3, 0).wait()
    compute_chunk(NCHUNK - 1, 0)
    out_copy(NCHUNK - 2, 1).wait()
    out_copy(NCHUNK - 1, 0).wait()


@functools.partial(jax.jit, static_argnames=())
def _sc_call(weights, z_samp, u, interval_interp):
    mesh = plsc.VectorSubcoreMesh(
        core_axis_name="c", subcore_axis_name="s", num_cores=NC, num_subcores=NS
    )
    fb = jnp.float32
    fn = pl.kernel(
        _sc_body,
        out_type=jax.ShapeDtypeStruct((B * K,), fb),
        mesh=mesh,
        scratch_types=[pltpu.VMEM((CHUNK_ELEMS,), fb) for _ in range(10)]
        + [pltpu.SemaphoreType.DMA for _ in range(4)],
        compiler_params=pltpu.CompilerParams(needs_layout_passes=False),
    )
    return fn(weights, z_samp, u, interval_interp)


def kernel(rays, weights, z_samp, u, interval_interp):
    del rays
    out = _sc_call(
        weights.reshape(-1),
        z_samp.reshape(-1),
        u.reshape(-1),
        interval_interp.reshape(-1),
    )
    return out.reshape(B, K)
